# Initial kernel scaffold; baseline (speedup 1.0000x reference)
import jax
import jax.numpy as jnp
from jax.experimental import pallas as pl


def kernel(xyz, cls_label, params):
    raise NotImplementedError("write your pallas kernel here")


# speedup vs baseline: 7.1577x; 7.1577x over previous
import functools
import jax
import jax.numpy as jnp
from jax import lax
from jax.experimental import pallas as pl
from jax.experimental.pallas import tpu as pltpu
from jax.experimental.pallas import tpu_sc as plsc


def _fps_body(npoint, n, xyz_ref, out_ref):
    x = xyz_ref[...]
    b = x.shape[0]
    iota = lax.broadcasted_iota(jnp.int32, (b, 1, n), 2)

    def body(i, carry):
        dist, far = carry
        oh = (iota == far).astype(jnp.float32)
        c = jnp.sum(x * oh, axis=2)
        out_ref[:, pl.ds(i, 1), :] = c[:, None, :]
        d = jnp.sum((x - c[:, :, None]) ** 2, axis=1, keepdims=True)
        dist = jnp.minimum(dist, d)
        mx = jnp.max(dist, axis=2, keepdims=True)
        far = jnp.min(jnp.where(dist == mx, iota, n), axis=2, keepdims=True)
        return dist, far

    dist0 = jnp.full((b, 1, n), 1e10, jnp.float32)
    far0 = jnp.zeros((b, 1, 1), jnp.int32)
    lax.fori_loop(0, npoint, body, (dist0, far0))


def _fps(xyz_cn, npoint):
    b, _, n = xyz_cn.shape
    return pl.pallas_call(
        functools.partial(_fps_body, npoint, n),
        out_shape=jax.ShapeDtypeStruct((b, npoint, 3), jnp.float32),
    )(xyz_cn)


def _bq_body(radii, ks, n, s, tab_off, tab_stride, nxyz_ref, xyz_ref,
             *out_and_scratch):
    outs = out_and_scratch[:len(radii)]
    scr = out_and_scratch[-1]
    bidx = pl.program_id(0)
    x1 = xyz_ref[0]
    x2 = nxyz_ref[0]
    sq1 = jnp.sum(x1 ** 2, axis=1, keepdims=True)
    sq2 = jnp.sum(x2 ** 2, axis=1, keepdims=True)
    prod = jax.lax.dot_general(x1, x2, (((1,), (1,)), ((), ())),
                               preferred_element_type=jnp.float32)
    sqr = sq1 + jnp.transpose(sq2, (1, 0)) - 2.0 * prod
    fn = jnp.float32(n)
    iota = lax.broadcasted_iota(jnp.int32, (n, s), 0).astype(jnp.float32)
    for r_i, (radius, k) in enumerate(zip(radii, ks)):
        v0 = jnp.where(sqr > radius * radius, fn, iota)

        def body(j, carry):
            v, first = carry
            m = jnp.min(v, axis=0, keepdims=True)
            v = jnp.where(v == m, fn, v)
            first = jnp.where(j == 0, m, first)
            mc = jnp.where(m == fn, first, m)
            scr[pl.ds(j, 1), :] = mc
            return v, first

        lax.fori_loop(0, k, body, (v0, jnp.zeros((1, s), jnp.float32)))
        g = jnp.transpose(scr[pl.ds(0, k), :], (1, 0))
        gi = jnp.clip(g.astype(jnp.int32), 0, n - 1)
        outs[r_i][0] = gi + (bidx * tab_stride + tab_off[r_i] * n)


def _ball_query(new_xyz, xyz_t, radii, ks, tab_off, tab_stride):
    b, s, _ = new_xyz.shape
    n = xyz_t.shape[1]
    kmax = max(ks)
    return pl.pallas_call(
        functools.partial(_bq_body, tuple(radii), tuple(ks), n, s,
                          tuple(tab_off), tab_stride),
        grid=(b,),
        in_specs=[pl.BlockSpec((1, s, 3), lambda i: (i, 0, 0)),
                  pl.BlockSpec((1, n, 3), lambda i: (i, 0, 0))],
        out_specs=[pl.BlockSpec((1, s, k), lambda i: (i, 0, 0)) for k in ks],
        out_shape=[jax.ShapeDtypeStruct((b, s, k), jnp.int32) for k in ks],
        scratch_shapes=[pltpu.VMEM((kmax, s), jnp.float32)],
    )(new_xyz, xyz_t)


def _sc_gather(table, idx):
    m0, = idx.shape
    t, c = table.shape
    info = plsc.get_sparse_core_info()
    nw = info.num_cores * info.num_subcores
    assert c % 16 == 0, c
    m = ((m0 + nw * 128 - 1) // (nw * 128)) * (nw * 128)
    if m != m0:
        idx = jnp.pad(idx, (0, m - m0))
    chunks = m // 128
    per_w = chunks // nw
    gpb = 1
    for cand in (8, 4, 2, 1):
        if per_w % cand == 0 and cand * 128 * c * 4 <= 360_000:
            gpb = cand
            break
    n_outer = per_w // gpb
    idx2 = idx.reshape(chunks, 128)
    mesh = plsc.VectorSubcoreMesh(core_axis_name="c", subcore_axis_name="s")

    @functools.partial(
        pl.kernel, mesh=mesh,
        compiler_params=pltpu.CompilerParams(use_tc_tiling_on_sc=False),
        out_type=jax.ShapeDtypeStruct((chunks, 128, c), jnp.float32),
        scratch_types=[
            pltpu.VMEM((gpb, 128), jnp.int32),
            pltpu.VMEM((gpb, 128, c), jnp.float32),
            pltpu.SemaphoreType.DMA,
        ],
    )
    def k(table_hbm, idx_hbm, out_hbm, idx_v, rows_v, sem):
        wid = lax.axis_index("s") * info.num_cores + lax.axis_index("c")
        base = wid * per_w

        def body(g, carry):
            row0 = base + g * gpb
            pltpu.sync_copy(idx_hbm.at[pl.ds(row0, gpb)], idx_v)
            cps = []
            for j in range(gpb):
                cps.append(pltpu.async_copy(
                    table_hbm.at[idx_v.at[j]], rows_v.at[j], sem))
            for cp in cps:
                cp.wait()
            pltpu.sync_copy(rows_v, out_hbm.at[pl.ds(row0, gpb)])
            return carry

        lax.fori_loop(0, n_outer, body, 0)

    out = k(table, idx2).reshape(m, c)
    return out[:m0] if m != m0 else out


def _bn_relu(y, stats, cnt, g, be):
    m = stats[0:1, :]
    v = stats[1:2, :] / cnt
    return jnp.maximum(g * (y - m) / jnp.sqrt(v + 1e-5) + be, 0.0)


def _welford_update(stats_ref, y2, rows):
    mu = jnp.sum(y2, axis=0, keepdims=True) / rows
    m2 = jnp.sum((y2 - mu) ** 2, axis=0, keepdims=True)
    pid = pl.program_id(0)

    @pl.when(pid == 0)
    def _():
        stats_ref[...] = jnp.concatenate([mu, m2], axis=0)

    @pl.when(pid != 0)
    def _():
        n_old = pid.astype(jnp.float32) * rows
        n_new = n_old + rows
        mean_acc = stats_ref[0:1, :]
        delta = mu - mean_acc
        stats_ref[0:1, :] = mean_acc + delta * (rows / n_new)
        stats_ref[1:2, :] += m2 + delta * delta * (n_old * rows / n_new)


def _y1_block(cin, gath_ref, nxyz_ref, w1p_ref, b1_ref):
    gath = gath_ref[...]
    sb, k, cpad = gath.shape
    nx = nxyz_ref[...]
    col = lax.broadcasted_iota(jnp.int32, (1, cpad), 1)
    cvec = jnp.zeros((sb, cpad), jnp.float32)
    for j in range(3):
        cvec = cvec + jnp.where(col == cin + j, nx[:, j:j + 1], 0.0)
    adj = gath - cvec[:, None, :]
    y = jax.lax.dot_general(
        adj.reshape(sb * k, cpad), w1p_ref[...],
        (((1,), (1,)), ((), ())), preferred_element_type=jnp.float32)
    c1 = y.shape[-1]
    return y.reshape(sb, k, c1) + b1_ref[...]


def _stats1_body(cnt, cin, *args):
    (gath_ref, nxyz_ref, w1p_ref, b1_ref, sums_ref) = args
    y = _y1_block(cin, gath_ref, nxyz_ref, w1p_ref, b1_ref)
    sb, k, c1 = y.shape
    _welford_update(sums_ref, y.reshape(sb * k, c1), float(sb * k))


def _mid_body(cnt, cin, *args):
    (gath_ref, nxyz_ref, w1p_ref, b1_ref, sums1_ref, g1_ref,
     be1_ref, w2_ref, b2_ref, y2_ref, sums2_ref) = args
    y1 = _y1_block(cin, gath_ref, nxyz_ref, w1p_ref, b1_ref)
    sb, k, c1 = y1.shape
    z1 = _bn_relu(y1, sums1_ref[...], cnt, g1_ref[...], be1_ref[...])
    y2 = jax.lax.dot_general(z1.reshape(sb * k, c1), w2_ref[...],
                             (((1,), (1,)), ((), ())),
                             preferred_element_type=jnp.float32) + b2_ref[...]
    c2 = y2.shape[-1]
    y2_ref[...] = y2.reshape(sb, k, c2)
    _welford_update(sums2_ref, y2, float(sb * k))


def _last_body(cnt, *args):
    (y2_ref, sums2_ref, g2_ref, be2_ref, w3_ref, b3_ref,
     m3_ref, sums3_ref) = args
    y2 = y2_ref[...]
    sb, k, c2 = y2.shape
    z2 = _bn_relu(y2, sums2_ref[...], cnt, g2_ref[...], be2_ref[...])
    y3 = jax.lax.dot_general(z2.reshape(sb * k, c2), w3_ref[...],
                             (((1,), (1,)), ((), ())),
                             preferred_element_type=jnp.float32) + b3_ref[...]
    c3 = y3.shape[-1]
    _welford_update(sums3_ref, y3, float(sb * k))
    m3_ref[...] = jnp.max(y3.reshape(sb, k, c3), axis=1)


def _full_spec(arr):
    nd = arr.ndim
    return pl.BlockSpec(arr.shape, lambda i, _n=nd: (0,) * _n)


def _grouped_branch(gath, new_xyz_f, w1p, b1, bn1, layer2, layer3,
                    r_rows, k, cin):
    cnt = float(r_rows * k)
    sb = max(8, 2048 // k)
    while r_rows % sb:
        sb //= 2
    grid = r_rows // sb
    cg = gath.shape[2]
    c1 = b1.shape[0]
    g1, be1 = bn1
    w2, b2, g2, be2 = layer2
    w3, b3, g3, be3 = layer3
    c2, c3 = w2.shape[0], w3.shape[0]

    row = lambda v: v.reshape(1, -1)
    base_in = [gath, new_xyz_f, w1p, row(b1)]
    base_specs = [
        pl.BlockSpec((sb, k, cg), lambda i: (i, 0, 0)),
        pl.BlockSpec((sb, 3), lambda i: (i, 0)),
    ] + [_full_spec(a) for a in base_in[2:]]

    sums1 = pl.pallas_call(
        functools.partial(_stats1_body, cnt, cin),
        grid=(grid,),
        in_specs=base_specs,
        out_specs=pl.BlockSpec((2, c1), lambda i: (0, 0)),
        out_shape=jax.ShapeDtypeStruct((2, c1), jnp.float32),
    )(*base_in)

    mid_extra = [sums1, row(g1), row(be1), w2, row(b2)]
    y2, sums2 = pl.pallas_call(
        functools.partial(_mid_body, cnt, cin),
        grid=(grid,),
        in_specs=base_specs + [_full_spec(a) for a in mid_extra],
        out_specs=[pl.BlockSpec((sb, k, c2), lambda i: (i, 0, 0)),
                   pl.BlockSpec((2, c2), lambda i: (0, 0))],
        out_shape=[jax.ShapeDtypeStruct((r_rows, k, c2), jnp.float32),
                   jax.ShapeDtypeStruct((2, c2), jnp.float32)],
    )(*(base_in + mid_extra))

    last_in = [y2, sums2, row(g2), row(be2), w3, row(b3)]
    m3, sums3 = pl.pallas_call(
        functools.partial(_last_body, cnt),
        grid=(grid,),
        in_specs=[pl.BlockSpec((sb, k, c2), lambda i: (i, 0, 0))]
        + [_full_spec(a) for a in last_in[1:]],
        out_specs=[pl.BlockSpec((sb, c3), lambda i: (i, 0)),
                   pl.BlockSpec((2, c3), lambda i: (0, 0))],
        out_shape=[jax.ShapeDtypeStruct((r_rows, c3), jnp.float32),
                   jax.ShapeDtypeStruct((2, c3), jnp.float32)],
    )(*last_in)
    return m3, sums3, (row(g3), row(be3))


def _finish_body(cnt, x_ref, sums_ref, g_ref, be_ref, out_ref):
    out_ref[...] = _bn_relu(x_ref[...], sums_ref[...], cnt,
                            g_ref[...], be_ref[...])


def _finish(x, sums, g, be, cnt):
    return pl.pallas_call(
        functools.partial(_finish_body, cnt),
        out_shape=jax.ShapeDtypeStruct(x.shape, jnp.float32),
    )(x, sums, g, be)


def _flat_body(layers_meta, mode, bsz, x_ref, *rest):
    out_ref = rest[-1]
    prefs = rest[:-1]
    x = x_ref[...]
    rows = x.shape[0]
    nl = len(layers_meta)
    for li in range(nl):
        w_ref, b_ref, g_ref, be_ref = prefs[4 * li:4 * li + 4]
        y = jax.lax.dot_general(x, w_ref[...], (((1,), (1,)), ((), ())),
                                preferred_element_type=jnp.float32) + b_ref[...]
        if mode == "nobn":
            x = y
            continue
        mu = jnp.sum(y, axis=0, keepdims=True) / rows
        m2 = jnp.sum((y - mu) ** 2, axis=0, keepdims=True)
        stats = jnp.concatenate([mu, m2], axis=0)
        if li == nl - 1 and mode == "max":
            c = y.shape[-1]
            mx = jnp.max(y.reshape(bsz, rows // bsz, c), axis=1)
            out_ref[...] = _bn_relu(mx, stats, float(rows),
                                    g_ref[...], be_ref[...])
            return
        x = _bn_relu(y, stats, float(rows), g_ref[...], be_ref[...])
    out_ref[...] = x


def _flat_mlp(x, layers, mode="plain", bsz=None):
    rows = x.shape[0]
    cout = layers[-1][0].shape[0]
    out_rows = bsz if mode == "max" else rows
    args = [x]
    meta = []
    for (w, b, g, be) in layers:
        args += [w, b.reshape(1, -1), g.reshape(1, -1), be.reshape(1, -1)]
        meta.append(w.shape)
    return pl.pallas_call(
        functools.partial(_flat_body, tuple(meta), mode, bsz),
        out_shape=jax.ShapeDtypeStruct((out_rows, cout), jnp.float32),
    )(*args)


def _head_body(x_ref, w1_ref, b1_ref, g1_ref, be1_ref, w2_ref, b2_ref,
               out_ref):
    x = x_ref[...]
    rows = x.shape[0]
    y = jax.lax.dot_general(x, w1_ref[...], (((1,), (1,)), ((), ())),
                            preferred_element_type=jnp.float32) + b1_ref[...]
    mu = jnp.sum(y, axis=0, keepdims=True) / rows
    m2 = jnp.sum((y - mu) ** 2, axis=0, keepdims=True)
    z = _bn_relu(y, jnp.concatenate([mu, m2], 0), float(rows),
                 g1_ref[...], be1_ref[...])
    lg = jax.lax.dot_general(z, w2_ref[...], (((1,), (1,)), ((), ())),
                             preferred_element_type=jnp.float32) + b2_ref[...]
    m = jnp.max(lg, axis=1, keepdims=True)
    e = jnp.exp(lg - m)
    lse = jnp.log(jnp.sum(e, axis=1, keepdims=True))
    out_ref[...] = lg - m - lse


def _head(x, conv1, w2, b2):
    rows = x.shape[0]
    w1, bb1, g1, be1 = conv1
    return pl.pallas_call(
        _head_body,
        out_shape=jax.ShapeDtypeStruct((rows, w2.shape[0]), jnp.float32),
    )(x, w1, bb1.reshape(1, -1), g1.reshape(1, -1), be1.reshape(1, -1),
      w2, b2.reshape(1, -1))


def _knn_body(n, s, x1_ref, x2_ref, idx_ref, w_ref):
    x1 = x1_ref[0]
    x2 = x2_ref[0]
    bidx = pl.program_id(0)
    sq1 = jnp.sum(x1 ** 2, axis=1, keepdims=True)
    sq2 = jnp.sum(x2 ** 2, axis=1, keepdims=True)
    prod = jax.lax.dot_general(x1, x2, (((1,), (1,)), ((), ())),
                               preferred_element_type=jnp.float32)
    d = sq1 + jnp.transpose(sq2, (1, 0)) - 2.0 * prod
    iota = lax.broadcasted_iota(jnp.int32, (n, s), 1).astype(jnp.float32)
    fs = jnp.float32(s)
    dd = d
    idxs, recs = [], []
    for _ in range(3):
        m = jnp.min(dd, axis=1, keepdims=True)
        idx = jnp.min(jnp.where(dd == m, iota, fs), axis=1, keepdims=True)
        oh = iota == idx
        idxs.append(idx)
        recs.append(1.0 / (m + 1e-8))
        dd = jnp.where(oh, 1e30, dd)
    rec_sum = (recs[0] + recs[1]) + recs[2]
    idx_ref[0] = (jnp.concatenate(idxs, axis=1).astype(jnp.int32)
                  + bidx * s)
    w_ref[0] = jnp.concatenate(recs, axis=1) / rec_sum


def _knn3(x1, x2):
    b, n, _ = x1.shape
    s = x2.shape[1]
    return pl.pallas_call(
        functools.partial(_knn_body, n, s),
        grid=(b,),
        in_specs=[pl.BlockSpec((1, n, 3), lambda i: (i, 0, 0)),
                  pl.BlockSpec((1, s, 3), lambda i: (i, 0, 0))],
        out_specs=[pl.BlockSpec((1, n, 3), lambda i: (i, 0, 0)),
                   pl.BlockSpec((1, n, 3), lambda i: (i, 0, 0))],
        out_shape=[jax.ShapeDtypeStruct((b, n, 3), jnp.int32),
                   jax.ShapeDtypeStruct((b, n, 3), jnp.float32)],
    )(x1, x2)


def _combine_body(g3_ref, w_ref, out_ref):
    g = g3_ref[...]
    w = w_ref[...]
    t0 = g[:, 0, :] * w[:, 0:1]
    t1 = g[:, 1, :] * w[:, 1:2]
    t2 = g[:, 2, :] * w[:, 2:3]
    out_ref[...] = (t0 + t1) + t2


def _knn_combine(g3, w):
    r, _, c = g3.shape
    rb = 512
    while r % rb:
        rb //= 2
    return pl.pallas_call(
        _combine_body,
        grid=(r // rb,),
        in_specs=[pl.BlockSpec((rb, 3, c), lambda i: (i, 0, 0)),
                  pl.BlockSpec((rb, 3), lambda i: (i, 0))],
        out_specs=pl.BlockSpec((rb, c), lambda i: (i, 0)),
        out_shape=jax.ShapeDtypeStruct((r, c), jnp.float32),
    )(g3, w)


def _knn_interp(x1, x2, p2):
    b, n, _ = x1.shape
    s = x2.shape[1]
    c = p2.shape[2]
    idx, w = _knn3(x1, x2)
    g3 = _sc_gather(p2.reshape(b * s, c), idx.reshape(-1))
    interp = _knn_combine(g3.reshape(b * n, 3, c), w.reshape(b * n, 3))
    return interp.reshape(b, n, c)


def _sa_msg(xyz_cn, xyz_t, pts, npoint, radii, ks, branches):
    b, n, _ = xyz_t.shape
    new_xyz = _fps(xyz_cn, npoint)
    nb = len(radii)
    raw = jnp.concatenate([pts, xyz_t], -1) if pts is not None else \
        jnp.concatenate([xyz_t, xyz_t], -1)
    cin = raw.shape[-1] - 3
    cpad = ((raw.shape[-1] + 15) // 16) * 16
    table = jnp.pad(raw, ((0, 0), (0, 0), (0, cpad - raw.shape[-1])))
    table = table.reshape(b * n, cpad)

    grps = _ball_query(new_xyz, xyz_t, radii, ks, [0] * nb, n)
    idx_flat = jnp.concatenate([g.reshape(-1) for g in grps])
    gath_all = _sc_gather(table, idx_flat)
    r_rows = b * npoint
    nxyz_f = new_xyz.reshape(r_rows, 3)

    m3s, sums3s, gs, bes = [], [], [], []
    start = 0
    for i, br in enumerate(branches):
        k = ks[i]
        seg = gath_all[start:start + r_rows * k]
        start += r_rows * k
        gath = seg.reshape(r_rows, k, -1)
        w1, b1, g1, be1 = br[0]
        w1p = jnp.pad(w1, ((0, 0), (0, cpad - w1.shape[1])))
        m3, sums3, (g3r, be3r) = _grouped_branch(
            gath, nxyz_f, w1p, b1, (g1, be1), br[1], br[2],
            r_rows, k, cin)
        m3s.append(m3)
        sums3s.append(sums3 * jnp.array([[1.0], [1.0 / (r_rows * k)]],
                                        jnp.float32))
        gs.append(g3r)
        bes.append(be3r)
    mcat = jnp.concatenate(m3s, axis=1)
    scat = jnp.concatenate(sums3s, axis=1)
    gcat = jnp.concatenate(gs, axis=1)
    becat = jnp.concatenate(bes, axis=1)
    out = _finish(mcat, scat, gcat, becat, 1.0)
    return new_xyz, out.reshape(b, npoint, -1)


def kernel(xyz, cls_label, params):
    b, _, n = xyz.shape
    xyz_t = xyz.transpose(0, 2, 1)

    l1_xyz, l1_pts = _sa_msg(xyz, xyz_t, None, 512,
                             [0.03, 0.06, 0.12], [32, 64, 128], params['sa1'])
    l2_xyz, l2_pts = _sa_msg(l1_xyz.transpose(0, 2, 1), l1_xyz, l1_pts, 256,
                             [0.12, 0.24], [64, 128], params['sa2'])
    l3_xyz, l3_pts = _sa_msg(l2_xyz.transpose(0, 2, 1), l2_xyz, l2_pts, 128,
                             [0.24, 0.48], [32, 64], params['sa3'])

    s3 = l3_xyz.shape[1]
    feats4 = jnp.concatenate([l3_xyz, l3_pts], -1).reshape(b * s3, -1)
    l4_pts = _flat_mlp(feats4, params['sa4'], mode="max", bsz=b)

    interp4 = jnp.broadcast_to(l4_pts[:, None, :], (b, s3, l4_pts.shape[1]))
    npts4 = jnp.concatenate([l3_pts, interp4], -1).reshape(b * s3, -1)
    l3_new = _flat_mlp(npts4, params['fp4']).reshape(b, s3, -1)

    s2 = l2_xyz.shape[1]
    interp3 = _knn_interp(l2_xyz, l3_xyz, l3_new)
    npts3 = jnp.concatenate([l2_pts, interp3], -1).reshape(b * s2, -1)
    l2_new = _flat_mlp(npts3, params['fp3']).reshape(b, s2, -1)

    s1 = l1_xyz.shape[1]
    interp2 = _knn_interp(l1_xyz, l2_xyz, l2_new)
    npts2 = jnp.concatenate([l1_pts, interp2], -1).reshape(b * s1, -1)
    l1_new = _flat_mlp(npts2, params['fp2']).reshape(b, s1, -1)

    interp1 = _knn_interp(xyz_t, l1_xyz, l1_new)
    cls_oh = jnp.broadcast_to(cls_label.reshape(b, 1, 1), (b, n, 1))
    npts1 = jnp.concatenate([cls_oh, xyz_t, xyz_t, interp1], -1)
    l0_new = _flat_mlp(npts1.reshape(b * n, -1), params['fp1'])

    w2, b2 = params['conv2']
    logits = _head(l0_new, params['conv1'], w2, b2)
    return logits.reshape(b, n, -1), l3_new.transpose(0, 2, 1)
